# baseline (speedup 1.0000x reference)
import math

import jax
import jax.numpy as jnp
from jax.experimental import pallas as pl
from jax.experimental.pallas import tpu as pltpu

N = 2048
D = 128
H = 8
DH = D // H
L = 2
FF = 256
NRBF = 8
NAA = 21
BLK = 256
BLKJ = 2048
IB = N // BLK
JB = N // BLKJ
EHW = 2 * DH

_EXPC = [math.exp(-8.0 * (2.0 * c / (NRBF - 1)) ** 2) for c in range(NRBF)]


def _ln(h):
    mu = jnp.mean(h, axis=-1, keepdims=True)
    hc = h - mu
    var = jnp.mean(hc * hc, axis=-1, keepdims=True)
    return hc / jnp.sqrt(var + 1e-5)


def _prep_body(apx_ref, apy_ref, apz_ref, mask_ref, aa_ref, ch_ref, wx_ref,
               wy_ref, wz_ref, emb_ref, bpos_ref, h0_ref, a_ref, b_ref):
    m = (mask_ref[...] > 0.5).astype(jnp.float32)
    apx = apx_ref[...]
    apy = apy_ref[...]
    apz = apz_ref[...]
    cax = apx[:, 1:2]
    cay = apy[:, 1:2]
    caz = apz[:, 1:2]
    h = jnp.dot((apx - cax) * m, wx_ref[...], preferred_element_type=jnp.float32)
    h += jnp.dot((apy - cay) * m, wy_ref[...], preferred_element_type=jnp.float32)
    h += jnp.dot((apz - caz) * m, wz_ref[...], preferred_element_type=jnp.float32)
    ai = jnp.clip(aa_ref[...].astype(jnp.int32), 0, NAA - 1)
    iota = jax.lax.broadcasted_iota(jnp.int32, (N, NAA), 1)
    onehot = (ai == iota).astype(jnp.float32)
    h += jnp.dot(onehot, emb_ref[...], preferred_element_type=jnp.float32)
    h0_ref[...] = h + bpos_ref[...]
    ch = ch_ref[...]
    n2 = cax * cax + cay * cay + caz * caz
    c2 = ch * ch
    ones = jnp.ones_like(n2)
    zeros = jnp.zeros_like(n2)
    a_ref[...] = jnp.concatenate(
        [-2.0 * cax, -2.0 * cay, -2.0 * caz, n2, ones,
         c2, -2.0 * ch, ones,
         zeros, zeros, zeros, zeros, zeros, zeros, zeros, zeros], axis=1)
    b_ref[...] = jnp.concatenate(
        [cax, cay, caz, ones, n2,
         ones, ch, c2,
         zeros, zeros, zeros, zeros, zeros, zeros, zeros, zeros], axis=1)


def _pre_body(h_ref, wq_ref, wk_ref, wv_ref, wpair_ref,
              qext_ref, kext_ref, vext_ref):
    hn = _ln(h_ref[...])
    q = jnp.dot(hn, wq_ref[...], preferred_element_type=jnp.float32)
    k = jnp.dot(hn, wk_ref[...], preferred_element_type=jnp.float32)
    v = jnp.dot(hn, wv_ref[...], preferred_element_type=jnp.float32)
    i0 = jax.lax.broadcasted_iota(jnp.int32, (D, H), 0) // DH
    i1 = jax.lax.broadcasted_iota(jnp.int32, (D, H), 1)
    sel = (i0 == i1).astype(jnp.float32)
    nq = jnp.sqrt(jnp.dot(q * q, sel, preferred_element_type=jnp.float32))
    nk2 = jnp.dot(k * k, sel, preferred_element_type=jnp.float32)
    kmax = jnp.sqrt(jnp.max(nk2, axis=0, keepdims=True))
    bbs = []
    for h in range(H):
        s = jnp.abs(wpair_ref[NRBF, h])
        for c in range(NRBF):
            s += jnp.abs(wpair_ref[c, h])
        bbs.append(s.reshape(1, 1))
    bb = jnp.concatenate(bbs, axis=1)
    mb = 0.25 * nq * kmax + bb
    j0 = jax.lax.broadcasted_iota(jnp.int32, (D, H * EHW), 0)
    j1 = jax.lax.broadcasted_iota(jnp.int32, (D, H * EHW), 1)
    head = j1 // EHW
    lane = j1 % EHW
    p1 = ((lane < DH) & (j0 == head * DH + lane)).astype(jnp.float32)
    m0 = jax.lax.broadcasted_iota(jnp.int32, (H, H * EHW), 0)
    m1 = jax.lax.broadcasted_iota(jnp.int32, (H, H * EHW), 1)
    pm = ((m1 % EHW == DH) & (m0 == m1 // EHW)).astype(jnp.float32)
    r1 = jax.lax.broadcasted_iota(jnp.int32, (1, H * EHW), 1)
    is_flag = (r1 % EHW == DH)
    negrow = jnp.where(is_flag, -1.0, 0.0)
    qext_ref[...] = (jnp.dot(q, p1 * 0.25, preferred_element_type=jnp.float32)
                     + jnp.dot(mb, pm, preferred_element_type=jnp.float32))
    kext_ref[...] = jnp.dot(k, p1, preferred_element_type=jnp.float32) + negrow
    vext_ref[...] = jnp.dot(v, p1, preferred_element_type=jnp.float32) - negrow


def _attn_body(a_ref, b_ref, qext_ref, kext_ref, vext_ref, wpair_ref, acc_ref):
    jb = pl.program_id(1)

    @pl.when(jb == 0)
    def _reset():
        acc_ref[...] = jnp.zeros((BLK, H * EHW), dtype=jnp.float32)

    a = a_ref[...]
    b = b_ref[...]
    d2 = jax.lax.dot_general(a[:, 0:5], b[:, 0:5], (((1,), (1,)), ((), ())),
                             preferred_element_type=jnp.float32)
    dc2 = jax.lax.dot_general(a[:, 5:8], b[:, 5:8], (((1,), (1,)), ((), ())),
                              preferred_element_type=jnp.float32)
    d2 = jnp.maximum(d2, 0.0) + 1e-8
    d = jnp.sqrt(d2)
    same = jnp.exp(dc2 * -30.0).astype(jnp.bfloat16)
    e0 = jnp.exp(d2 * -8.0).astype(jnp.bfloat16)
    u1 = jnp.exp(d * (32.0 / 7.0))
    u2 = u1 * u1
    u3 = u2 * u1
    u4 = u2 * u2
    u5 = u4 * u1
    u6 = u4 * u2
    u7 = u4 * u3
    ub = [None] + [x.astype(jnp.bfloat16) for x in (u1, u2, u3, u4, u5, u6, u7)]

    qx = qext_ref[...]
    kx = kext_ref[...]
    vt = vext_ref[...]
    for h in range(H):
        sl = slice(h * EHW, (h + 1) * EHW)
        logits = jax.lax.dot_general(qx[:, sl], kx[:, sl],
                                     (((1,), (1,)), ((), ())),
                                     preferred_element_type=jnp.float32)
        s = None
        for c in range(1, NRBF):
            ac = (wpair_ref[c, h] * _EXPC[c]).astype(jnp.bfloat16)
            s = ub[c] * ac if s is None else s + ub[c] * ac
        s += wpair_ref[0, h].astype(jnp.bfloat16)
        bias = s * e0 + same * wpair_ref[NRBF, h].astype(jnp.bfloat16)
        arg = logits + bias.astype(jnp.float32)
        p = jnp.exp(jnp.maximum(arg, -80.0))
        acc_ref[:, sl] += jnp.dot(p, vt[:, sl],
                                  preferred_element_type=jnp.float32)


def _ff_body(acc_ref, h_ref, wo_ref, w1_ref, w2_ref, out_ref):
    acc = acc_ref[...]
    j0 = jax.lax.broadcasted_iota(jnp.int32, (H * EHW, D), 0)
    j1 = jax.lax.broadcasted_iota(jnp.int32, (H * EHW, D), 1)
    head = j1 // DH
    pn = (j0 == head * EHW + (j1 % DH)).astype(jnp.float32)
    pd = (j0 == head * EHW + DH).astype(jnp.float32)
    num = jnp.dot(acc, pn, preferred_element_type=jnp.float32)
    den = jnp.dot(acc, pd, preferred_element_type=jnp.float32)
    o = num / den
    h1 = h_ref[...] + jnp.dot(o, wo_ref[...], preferred_element_type=jnp.float32)
    ffin = jax.nn.relu(jnp.dot(_ln(h1), w1_ref[...], preferred_element_type=jnp.float32))
    out_ref[...] = h1 + jnp.dot(ffin, w2_ref[...], preferred_element_type=jnp.float32)


def _attn_call(a_mat, b_mat, qext, kext, vext, wpair_l):
    return pl.pallas_call(
        _attn_body,
        grid=(IB, JB),
        in_specs=[
            pl.BlockSpec((BLK, 16), lambda i, j: (i, 0)),
            pl.BlockSpec((BLKJ, 16), lambda i, j: (j, 0)),
            pl.BlockSpec((BLK, H * EHW), lambda i, j: (i, 0)),
            pl.BlockSpec((BLKJ, H * EHW), lambda i, j: (j, 0)),
            pl.BlockSpec((BLKJ, H * EHW), lambda i, j: (j, 0)),
            pl.BlockSpec(memory_space=pltpu.SMEM),
        ],
        out_specs=pl.BlockSpec((BLK, H * EHW), lambda i, j: (i, 0)),
        out_shape=jax.ShapeDtypeStruct((N, H * EHW), jnp.float32),
        compiler_params=pltpu.CompilerParams(
            dimension_semantics=("arbitrary", "arbitrary")),
    )(a_mat, b_mat, qext, kext, vext, wpair_l)


def kernel(x, edge_index, edge_attr, aa_embed, w_pos, b_pos, wq, wk, wv, wo,
           w_pair, w1, w2):
    apx = x[:, 0:42:3]
    apy = x[:, 1:42:3]
    apz = x[:, 2:42:3]
    mask = x[:, 45:59]
    aa_col = x[:, 42:43]
    chf = x[:, 44:45].astype(jnp.int32).astype(jnp.float32)
    wx = w_pos[0::3]
    wy = w_pos[1::3]
    wz = w_pos[2::3]
    bpos = b_pos.reshape(1, D)

    h0, a_mat, b_mat = pl.pallas_call(
        _prep_body,
        out_shape=[
            jax.ShapeDtypeStruct((N, D), jnp.float32),
            jax.ShapeDtypeStruct((N, 16), jnp.float32),
            jax.ShapeDtypeStruct((N, 16), jnp.float32),
        ],
    )(apx, apy, apz, mask, aa_col, chf, wx, wy, wz, aa_embed, bpos)

    pre_call = pl.pallas_call(
        _pre_body,
        in_specs=[
            pl.BlockSpec((N, D), lambda: (0, 0)),
            pl.BlockSpec((D, D), lambda: (0, 0)),
            pl.BlockSpec((D, D), lambda: (0, 0)),
            pl.BlockSpec((D, D), lambda: (0, 0)),
            pl.BlockSpec(memory_space=pltpu.SMEM),
        ],
        out_specs=[
            pl.BlockSpec((N, H * EHW), lambda: (0, 0)),
            pl.BlockSpec((N, H * EHW), lambda: (0, 0)),
            pl.BlockSpec((N, H * EHW), lambda: (0, 0)),
        ],
        out_shape=[
            jax.ShapeDtypeStruct((N, H * EHW), jnp.float32),
            jax.ShapeDtypeStruct((N, H * EHW), jnp.float32),
            jax.ShapeDtypeStruct((N, H * EHW), jnp.float32),
        ],
    )

    ff_call = pl.pallas_call(
        _ff_body,
        out_shape=jax.ShapeDtypeStruct((N, D), jnp.float32),
    )

    h = h0
    for l in range(L):
        qext, kext, vext = pre_call(h, wq[l], wk[l], wv[l], w_pair[l])
        acc = _attn_call(a_mat, b_mat, qext, kext, vext, w_pair[l])
        h = ff_call(acc, h, wo[l], w1[l], w2[l])
    return h.reshape(1, N, D)
